# Initial kernel scaffold; baseline (speedup 1.0000x reference)
import jax
import jax.numpy as jnp
from jax.experimental import pallas as pl


def kernel(x, W_ft, b_ft, W_r1, b_r1, g_r, be_r, W_r2, b_r2, W_u, b_u, g_u, be_u):
    raise NotImplementedError("write your pallas kernel here")


# speedup vs baseline: 5.1701x; 5.1701x over previous
import functools

import jax
import jax.numpy as jnp
from jax import lax
from jax.experimental import pallas as pl
from jax.experimental.pallas import tpu as pltpu
from jax.experimental.pallas import tpu_sc as plsc

B, N, D, K, OUT = 4, 2048, 256, 9, 256
D2 = D // 2
EPS = 1e-5
BN_ROWS = B * N
M_EDGES = B * N * K

SC_NC, SC_NS = 2, 16
SC_NW = SC_NC * SC_NS
EDGES_PER_W = M_EDGES // SC_NW
GCHUNK = 128
NCHUNKS = EDGES_PER_W // GCHUNK


def _gelu(v):
    return 0.5 * v * (1.0 + lax.erf(v * 0.7071067811865476))


def _feat_body(x_ref, wft_ref, bft_ref, wr1_ref, xt_ref, p_ref):
    x = x_ref[...]
    xt_ref[...] = lax.dot_general(
        x, wft_ref[...], (((1,), (1,)), ((), ())),
        preferred_element_type=jnp.float32) + bft_ref[...]
    p_ref[...] = lax.dot_general(
        x, wr1_ref[...], (((1,), (1,)), ((), ())),
        preferred_element_type=jnp.float32)


def _feat(x2, W_ft, b_ft, W_r1):
    RB = 512
    grid = (BN_ROWS // RB,)
    return pl.pallas_call(
        _feat_body,
        grid=grid,
        in_specs=[
            pl.BlockSpec((RB, D), lambda i: (i, 0)),
            pl.BlockSpec((D, D), lambda i: (0, 0)),
            pl.BlockSpec((1, D), lambda i: (0, 0)),
            pl.BlockSpec((D2, D), lambda i: (0, 0)),
        ],
        out_specs=[
            pl.BlockSpec((RB, D), lambda i: (i, 0)),
            pl.BlockSpec((RB, D2), lambda i: (i, 0)),
        ],
        out_shape=[
            jax.ShapeDtypeStruct((BN_ROWS, D), jnp.float32),
            jax.ShapeDtypeStruct((BN_ROWS, D2), jnp.float32),
        ],
    )(x2, W_ft, b_ft, W_r1)


def _knn_body(a_ref, all_ref, idx_ref):
    RB = a_ref.shape[1]
    a = a_ref[0]
    ax = all_ref[0]
    xx_a = jnp.sum(a * a, axis=1, keepdims=True)
    xx_all = jnp.sum(ax * ax, axis=1)[None, :]
    xy = lax.dot_general(a, ax, (((1,), (1,)), ((), ())),
                         preferred_element_type=jnp.float32)
    d = xx_a + xx_all - 2.0 * xy
    iota = lax.broadcasted_iota(jnp.int32, (RB, N), 1)
    cols = []
    for t in range(K + 1):
        m = jnp.min(d, axis=1, keepdims=True)
        am = jnp.min(jnp.where(d == m, iota, jnp.int32(N)), axis=1,
                     keepdims=True)
        if t > 0:
            cols.append(am)
        d = jnp.where(iota == am, jnp.float32(jnp.inf), d)
    cols.append(jnp.zeros((RB, 16 - K), jnp.int32))
    idx = jnp.concatenate(cols, axis=1)
    idx_ref[0] = idx + pl.program_id(0) * N


def _knn(xt3):
    RB = 256
    grid = (B, N // RB)
    return pl.pallas_call(
        _knn_body,
        grid=grid,
        in_specs=[
            pl.BlockSpec((1, RB, D), lambda b, r: (b, r, 0)),
            pl.BlockSpec((1, N, D), lambda b, r: (b, 0, 0)),
        ],
        out_specs=pl.BlockSpec((1, RB, 16), lambda b, r: (b, r, 0)),
        out_shape=jax.ShapeDtypeStruct((B, N, 16), jnp.int32),
    )(xt3, xt3)


def _gather_sc_body(p_hbm, x_hbm, idx_hbm, gp_hbm, gx_hbm,
                    idx_v, prow_v, xrow_v, sem_p, sem_x):
    wid = lax.axis_index("s") * SC_NC + lax.axis_index("c")
    for c in range(NCHUNKS):
        base = wid * EDGES_PER_W + c * GCHUNK
        pltpu.sync_copy(idx_hbm.at[pl.ds(base, GCHUNK)], idx_v)
        cp_p = pltpu.async_copy(p_hbm.at[idx_v], prow_v, sem_p)
        cp_x = pltpu.async_copy(x_hbm.at[idx_v], xrow_v, sem_x)
        cp_p.wait()
        cp_x.wait()
        pltpu.sync_copy(prow_v, gp_hbm.at[pl.ds(base, GCHUNK)])
        pltpu.sync_copy(xrow_v, gx_hbm.at[pl.ds(base, GCHUNK)])


@functools.lru_cache(maxsize=1)
def _make_gather_sc():
    return pl.kernel(
        _gather_sc_body,
        out_type=[
            jax.ShapeDtypeStruct((M_EDGES, D2), jnp.float32),
            jax.ShapeDtypeStruct((M_EDGES, D), jnp.float32),
        ],
        mesh=plsc.VectorSubcoreMesh(core_axis_name="c", subcore_axis_name="s",
                                    num_cores=SC_NC, num_subcores=SC_NS),
        scratch_types=[
            pltpu.VMEM((GCHUNK,), jnp.int32),
            pltpu.VMEM((GCHUNK, D2), jnp.float32),
            pltpu.VMEM((GCHUNK, D), jnp.float32),
            pltpu.SemaphoreType.DMA,
            pltpu.SemaphoreType.DMA,
        ],
    )


def _gather_sc(p, x2, idx_flat):
    return _make_gather_sc()(p, x2, idx_flat)


def _stats_body(gp_ref, p_ref, out_ref):
    PB = p_ref.shape[0]
    y3 = gp_ref[...].reshape(PB, K, D2) - p_ref[...][:, None, :]
    ym = y3.reshape(PB * K, D2)
    s1 = jnp.sum(ym, axis=0, keepdims=True)
    s2 = jnp.sum(ym * ym, axis=0, keepdims=True)

    @pl.when(pl.program_id(0) == 0)
    def _():
        out_ref[...] = jnp.zeros_like(out_ref)

    out_ref[0:1, :] += s1
    out_ref[1:2, :] += s2


def _stats(gp, p):
    PB = 512
    grid = (BN_ROWS // PB,)
    return pl.pallas_call(
        _stats_body,
        grid=grid,
        in_specs=[
            pl.BlockSpec((PB * K, D2), lambda i: (i, 0)),
            pl.BlockSpec((PB, D2), lambda i: (i, 0)),
        ],
        out_specs=pl.BlockSpec((8, D2), lambda i: (0, 0)),
        out_shape=jax.ShapeDtypeStruct((8, D2), jnp.float32),
    )(gp, p)


def _agg_body(gp_ref, gx_ref, p_ref, x_ref, sums_ref, gr_ref, ber_ref,
              wr2_ref, br2_ref, wux_ref, wua_ref, bu_ref, u_ref, s2_ref):
    PB = x_ref.shape[0]
    cnt = jnp.float32(M_EDGES)
    mu = sums_ref[0:1, :] / cnt
    var = sums_ref[1:2, :] / cnt - mu * mu
    scale = gr_ref[...] * lax.rsqrt(var + EPS)
    shift = ber_ref[...] - mu * scale

    y3 = gp_ref[...].reshape(PB, K, D2) - p_ref[...][:, None, :]
    h = _gelu(y3 * scale[None] + shift[None])
    z = jnp.sum(h * wr2_ref[...][None], axis=2, keepdims=True)
    z = z + br2_ref[0:1, 0:1][None]
    w = 1.0 / (1.0 + jnp.exp(-z))

    ex = gx_ref[...].reshape(PB, K, D) - x_ref[...][:, None, :]
    agg = jnp.sum(w * ex, axis=1)

    u = (lax.dot_general(x_ref[...], wux_ref[...], (((1,), (1,)), ((), ())),
                         preferred_element_type=jnp.float32)
         + lax.dot_general(agg, wua_ref[...], (((1,), (1,)), ((), ())),
                           preferred_element_type=jnp.float32)
         + bu_ref[...])
    u_ref[...] = u

    @pl.when(pl.program_id(0) == 0)
    def _():
        s2_ref[...] = jnp.zeros_like(s2_ref)

    s2_ref[0:1, :] += jnp.sum(u, axis=0, keepdims=True)
    s2_ref[1:2, :] += jnp.sum(u * u, axis=0, keepdims=True)


def _agg(gp, gx, p, x2, sums, g_r2, be_r2, wr2, br2, wux, wua, bu):
    PB = 256
    grid = (BN_ROWS // PB,)
    return pl.pallas_call(
        _agg_body,
        grid=grid,
        in_specs=[
            pl.BlockSpec((PB * K, D2), lambda i: (i, 0)),
            pl.BlockSpec((PB * K, D), lambda i: (i, 0)),
            pl.BlockSpec((PB, D2), lambda i: (i, 0)),
            pl.BlockSpec((PB, D), lambda i: (i, 0)),
            pl.BlockSpec((8, D2), lambda i: (0, 0)),
            pl.BlockSpec((1, D2), lambda i: (0, 0)),
            pl.BlockSpec((1, D2), lambda i: (0, 0)),
            pl.BlockSpec((1, D2), lambda i: (0, 0)),
            pl.BlockSpec((1, D2), lambda i: (0, 0)),
            pl.BlockSpec((OUT, D), lambda i: (0, 0)),
            pl.BlockSpec((OUT, D), lambda i: (0, 0)),
            pl.BlockSpec((1, OUT), lambda i: (0, 0)),
        ],
        out_specs=[
            pl.BlockSpec((PB, OUT), lambda i: (i, 0)),
            pl.BlockSpec((8, OUT), lambda i: (0, 0)),
        ],
        out_shape=[
            jax.ShapeDtypeStruct((BN_ROWS, OUT), jnp.float32),
            jax.ShapeDtypeStruct((8, OUT), jnp.float32),
        ],
    )(gp, gx, p, x2, sums, g_r2, be_r2, wr2, br2, wux, wua, bu)


def _final_body(u_ref, s2_ref, gu_ref, beu_ref, o_ref):
    cnt = jnp.float32(BN_ROWS)
    mu = s2_ref[0:1, :] / cnt
    var = s2_ref[1:2, :] / cnt - mu * mu
    scale = gu_ref[...] * lax.rsqrt(var + EPS)
    shift = beu_ref[...] - mu * scale
    o_ref[...] = _gelu(u_ref[...] * scale + shift)


def _final(u, s2, g_u2, be_u2):
    RB = 512
    grid = (BN_ROWS // RB,)
    return pl.pallas_call(
        _final_body,
        grid=grid,
        in_specs=[
            pl.BlockSpec((RB, OUT), lambda i: (i, 0)),
            pl.BlockSpec((8, OUT), lambda i: (0, 0)),
            pl.BlockSpec((1, OUT), lambda i: (0, 0)),
            pl.BlockSpec((1, OUT), lambda i: (0, 0)),
        ],
        out_specs=pl.BlockSpec((RB, OUT), lambda i: (i, 0)),
        out_shape=jax.ShapeDtypeStruct((BN_ROWS, OUT), jnp.float32),
    )(u, s2, g_u2, be_u2)


def kernel(x, W_ft, b_ft, W_r1, b_r1, g_r, be_r, W_r2, b_r2, W_u, b_u, g_u, be_u):
    x2 = x.reshape(BN_ROWS, D)

    xt, p = _feat(x2, W_ft, b_ft.reshape(1, D), W_r1)

    idx16 = _knn(xt.reshape(B, N, D))
    idx_flat = idx16[:, :, :K].reshape(-1)

    gp, gx = _gather_sc(p, x2, idx_flat)

    sums = _stats(gp, p)

    wux = W_u[:, :D]
    wua = W_u[:, D:]
    u, s2 = _agg(gp, gx, p, x2, sums,
                 g_r.reshape(1, D2), be_r.reshape(1, D2),
                 W_r2.reshape(1, D2), jnp.broadcast_to(b_r2, (1, D2)),
                 wux, wua, b_u.reshape(1, OUT))

    out = _final(u, s2, g_u.reshape(1, OUT), be_u.reshape(1, OUT))
    return out.reshape(B, N, OUT)


# speedup vs baseline: 11.3526x; 2.1958x over previous
import functools

import jax
import jax.numpy as jnp
from jax import lax
from jax.experimental import pallas as pl
from jax.experimental.pallas import tpu as pltpu
from jax.experimental.pallas import tpu_sc as plsc

B, N, D, K, OUT = 4, 2048, 256, 9, 256
D2 = D // 2
EPS = 1e-5
BN_ROWS = B * N
M_EDGES = B * N * K

NH = 2
HB = B // NH
EDGES_H = M_EDGES // NH

SC_NC, SC_NS = 2, 16
SC_NW = SC_NC * SC_NS
EDGES_PER_W = EDGES_H // SC_NW
GCHUNK = 128
NCHUNKS = EDGES_PER_W // GCHUNK


def _gelu(v):
    return 0.5 * v * (1.0 + lax.erf(v * 0.7071067811865476))


def _round_bf16_bits(v):
    u = lax.bitcast_convert_type(v, jnp.int32)
    r = u + 32767 + jnp.bitwise_and(jnp.right_shift(u, 16), 1)
    return jnp.bitwise_and(r, jnp.int32(-65536))


def _pack_halves(v, w):
    lo = lax.slice(v, (0, 0), (v.shape[0], w // 2))
    hi = lax.slice(v, (0, w // 2), (v.shape[0], w))
    return jnp.bitwise_or(
        lax.shift_right_logical(_round_bf16_bits(lo), 16),
        _round_bf16_bits(hi))


def _unpack_halves(vi):
    fe = lax.bitcast_convert_type(jnp.left_shift(vi, 16), jnp.float32)
    fo = lax.bitcast_convert_type(
        jnp.bitwise_and(vi, jnp.int32(-65536)), jnp.float32)
    return jnp.concatenate([fe, fo], axis=1)


def _feat_body(x_ref, wft_ref, bft_ref, wr1_ref, xt_ref, p_ref, xx_ref,
               xp_ref):
    x = x_ref[...]
    xt = lax.dot_general(
        x, wft_ref[...], (((1,), (1,)), ((), ())),
        preferred_element_type=jnp.float32) + bft_ref[...]
    xt_ref[...] = xt
    p = lax.dot_general(
        x, wr1_ref[...], (((1,), (1,)), ((), ())),
        preferred_element_type=jnp.float32)
    p_ref[...] = p
    xx_ref[...] = jnp.sum(xt * xt, axis=1)[None, None, :]
    xp_ref[...] = jnp.concatenate(
        [lax.bitcast_convert_type(p, jnp.int32), _pack_halves(x, D)], axis=1)


def _feat(x2, W_ft, b_ft, W_r1):
    RB = 512
    grid = (BN_ROWS // RB,)
    nb = N // RB
    return pl.pallas_call(
        _feat_body,
        grid=grid,
        in_specs=[
            pl.BlockSpec((RB, D), lambda i: (i, 0)),
            pl.BlockSpec((D, D), lambda i: (0, 0)),
            pl.BlockSpec((1, D), lambda i: (0, 0)),
            pl.BlockSpec((D2, D), lambda i: (0, 0)),
        ],
        out_specs=[
            pl.BlockSpec((RB, D), lambda i: (i, 0)),
            pl.BlockSpec((RB, D2), lambda i: (i, 0)),
            pl.BlockSpec((1, 1, RB), lambda i: (i // nb, 0, i % nb)),
            pl.BlockSpec((RB, D), lambda i: (i, 0)),
        ],
        out_shape=[
            jax.ShapeDtypeStruct((BN_ROWS, D), jnp.float32),
            jax.ShapeDtypeStruct((BN_ROWS, D2), jnp.float32),
            jax.ShapeDtypeStruct((B, 1, N), jnp.float32),
            jax.ShapeDtypeStruct((BN_ROWS, D), jnp.int32),
        ],
    )(x2, W_ft, b_ft, W_r1)


def _knn_body(a_ref, all_ref, xx_ref, idx_ref, *, h):
    RB = a_ref.shape[1]
    a = a_ref[0]
    ax = all_ref[0]
    xx_a = jnp.sum(a * a, axis=1, keepdims=True)
    xx_all = xx_ref[0]
    xy = lax.dot_general(a, ax, (((1,), (1,)), ((), ())),
                         preferred_element_type=jnp.float32)
    d = xx_a + xx_all - 2.0 * xy
    iota = lax.broadcasted_iota(jnp.int32, (RB, N), 1)
    self_col = (lax.broadcasted_iota(jnp.int32, (RB, 1), 0)
                + pl.program_id(1) * RB)
    d = jnp.where(iota == self_col, jnp.float32(jnp.inf), d)

    NG, Q = 16, 4
    L = N // NG
    INF = jnp.float32(jnp.inf)
    iota_l = lax.broadcasted_iota(jnp.int32, (RB, L), 1).astype(jnp.float32)
    vs = [jnp.full((RB, L), INF, jnp.float32) for _ in range(Q)]
    cs = [jnp.zeros((RB, L), jnp.float32) for _ in range(Q)]
    for g in range(NG):
        val = lax.slice(d, (0, g * L), (RB, (g + 1) * L))
        col = iota_l + jnp.float32(g * L)
        for q in range(Q):
            lt = val < vs[q]
            nv = jnp.where(lt, val, vs[q])
            nc = jnp.where(lt, col, cs[q])
            val = jnp.where(lt, vs[q], val)
            col = jnp.where(lt, cs[q], col)
            vs[q], cs[q] = nv, nc
    cols = []
    for _ in range(K):
        m = jnp.min(vs[0], axis=1, keepdims=True)
        am = jnp.min(jnp.where(vs[0] == m, cs[0], jnp.float32(N)), axis=1,
                     keepdims=True)
        cols.append(am)
        hit = cs[0] == am
        for q in range(Q - 1):
            vs[q] = jnp.where(hit, vs[q + 1], vs[q])
            cs[q] = jnp.where(hit, cs[q + 1], cs[q])
        vs[Q - 1] = jnp.where(hit, INF, vs[Q - 1])
        cs[Q - 1] = jnp.where(hit, jnp.float32(0), cs[Q - 1])
    idx = (jnp.concatenate(cols, axis=1).astype(jnp.int32)
           + (h * HB + pl.program_id(0)) * N)
    idx_ref[...] = idx.T


def _knn(xt3, xx3, h):
    RB = 512
    grid = (HB, N // RB)
    return pl.pallas_call(
        functools.partial(_knn_body, h=h),
        grid=grid,
        in_specs=[
            pl.BlockSpec((1, RB, D), lambda b, r: (h * HB + b, r, 0)),
            pl.BlockSpec((1, N, D), lambda b, r: (h * HB + b, 0, 0)),
            pl.BlockSpec((1, 1, N), lambda b, r: (h * HB + b, 0, 0)),
        ],
        out_specs=pl.BlockSpec((K, RB), lambda b, r: (0, b * (N // RB) + r)),
        out_shape=jax.ShapeDtypeStruct((K, HB * N), jnp.int32),
    )(xt3, xt3, xx3)


def _gather_sc_body(t_hbm, idx_hbm, g_hbm,
                    idx_a, idx_b, row_a, row_b,
                    sem_ga, sem_gb, sem_wa, sem_wb):
    wid = lax.axis_index("s") * SC_NC + lax.axis_index("c")
    bufs = [(idx_a, row_a, sem_ga, sem_wa),
            (idx_b, row_b, sem_gb, sem_wb)]

    def start_gather(c):
        idx_v, row_v, sem_g, _ = bufs[c % 2]
        base = wid * EDGES_PER_W + c * GCHUNK
        pltpu.sync_copy(idx_hbm.at[pl.ds(base, GCHUNK)], idx_v)
        return (pltpu.async_copy(t_hbm.at[idx_v], row_v, sem_g),)

    def start_write(c):
        _, row_v, _, sem_w = bufs[c % 2]
        base = wid * EDGES_PER_W + c * GCHUNK
        return (pltpu.async_copy(row_v, g_hbm.at[pl.ds(base, GCHUNK)], sem_w),)

    pending_g = {0: start_gather(0)}
    pending_w = {}
    for c in range(NCHUNKS):
        if c + 1 < NCHUNKS:
            if c - 1 >= 0:
                for h in pending_w.pop(c - 1):
                    h.wait()
            pending_g[c + 1] = start_gather(c + 1)
        for h in pending_g.pop(c):
            h.wait()
        pending_w[c] = start_write(c)
    for c in (NCHUNKS - 2, NCHUNKS - 1):
        for h in pending_w.pop(c, ()):
            h.wait()


@functools.lru_cache(maxsize=1)
def _make_gather_sc():
    return pl.kernel(
        _gather_sc_body,
        out_type=jax.ShapeDtypeStruct((EDGES_H, D), jnp.int32),
        mesh=plsc.VectorSubcoreMesh(core_axis_name="c", subcore_axis_name="s",
                                    num_cores=SC_NC, num_subcores=SC_NS),
        scratch_types=[
            pltpu.VMEM((GCHUNK,), jnp.int32),
            pltpu.VMEM((GCHUNK,), jnp.int32),
            pltpu.VMEM((GCHUNK, D), jnp.int32),
            pltpu.VMEM((GCHUNK, D), jnp.int32),
            pltpu.SemaphoreType.DMA,
            pltpu.SemaphoreType.DMA,
            pltpu.SemaphoreType.DMA,
            pltpu.SemaphoreType.DMA,
        ],
    )


def _gather_sc(table, idx_flat):
    return _make_gather_sc()(table, idx_flat)


def _agg_stats_part(gp_refs, pv, st_ref):
    s1 = jnp.zeros((1, D2), jnp.float32)
    s2 = jnp.zeros((1, D2), jnp.float32)
    for k in range(K):
        y = lax.bitcast_convert_type(gp_refs[k][...], jnp.float32) - pv
        s1 = s1 + jnp.sum(y, axis=0, keepdims=True)
        s2 = s2 + jnp.sum(y * y, axis=0, keepdims=True)
    st_ref[0:1, :] += s1
    st_ref[1:2, :] += s2


def _agg_main_part(gp_refs, gx_refs, pv, x_ref, gr_ref, ber_ref, wr2_ref,
                   br2_ref, wu_ref, bu_ref, u_scr, s2_scr, st_ref, i):
    PB = pv.shape[0]
    cnt = jnp.float32(M_EDGES)
    mu = st_ref[0:1, :] / cnt
    var = st_ref[1:2, :] / cnt - mu * mu
    scale = gr_ref[...] * lax.rsqrt(var + EPS)
    shift = ber_ref[...] - mu * scale
    wr2 = wr2_ref[...]
    br2 = br2_ref[...]
    wu = wu_ref[...]
    wux = lax.slice(wu, (0, 0), (OUT, D))
    wua = lax.slice(wu, (0, D), (OUT, 2 * D))

    xv = x_ref[...]
    agg_e = jnp.zeros((PB, D2), jnp.float32)
    agg_o = jnp.zeros((PB, D2), jnp.float32)
    wsum = jnp.zeros((PB, 1), jnp.float32)
    for k in range(K):
        y = lax.bitcast_convert_type(gp_refs[k][...], jnp.float32) - pv
        h = _gelu(y * scale + shift)
        z = jnp.sum(h * wr2, axis=1, keepdims=True) + br2
        w = 1.0 / (1.0 + jnp.exp(-z))
        gxi = gx_refs[k][...]
        fe = lax.bitcast_convert_type(jnp.left_shift(gxi, 16), jnp.float32)
        fo = lax.bitcast_convert_type(
            jnp.bitwise_and(gxi, jnp.int32(-65536)), jnp.float32)
        agg_e = agg_e + w * fe
        agg_o = agg_o + w * fo
        wsum = wsum + w
    agg = jnp.concatenate([agg_e, agg_o], axis=1) - wsum * xv

    u = (lax.dot_general(xv, wux, (((1,), (1,)), ((), ())),
                         preferred_element_type=jnp.float32)
         + lax.dot_general(agg, wua, (((1,), (1,)), ((), ())),
                           preferred_element_type=jnp.float32)
         + bu_ref[...])
    u_scr[pl.ds(i * PB, PB), :] = u

    @pl.when(i == 0)
    def _():
        s2_scr[...] = jnp.zeros_like(s2_scr)

    s2_scr[0:1, :] += jnp.sum(u, axis=0, keepdims=True)
    s2_scr[1:2, :] += jnp.sum(u * u, axis=0, keepdims=True)


def _agg_body(*refs):
    gp0 = refs[:K]
    gp1 = refs[K:2 * K]
    gx0 = refs[2 * K:3 * K]
    gx1 = refs[3 * K:4 * K]
    (p_ref, x_ref, gr_ref, ber_ref, wr2_ref, br2_ref,
     wu_ref, bu_ref, gu_ref, beu_ref, o_ref,
     st_ref, u_scr, s2_scr) = refs[4 * K:]
    ph = pl.program_id(0)
    i = pl.program_id(1)
    nh = pl.num_programs(1) // NH
    pv = p_ref[...]
    PB = pv.shape[0]

    @pl.when(jnp.logical_and(ph == 0, i == 0))
    def _():
        st_ref[...] = jnp.zeros_like(st_ref)

    @pl.when(jnp.logical_and(ph == 0, i < nh))
    def _():
        _agg_stats_part(gp0, pv, st_ref)

    @pl.when(jnp.logical_and(ph == 0, i >= nh))
    def _():
        _agg_stats_part(gp1, pv, st_ref)

    @pl.when(jnp.logical_and(ph == 1, i < nh))
    def _():
        _agg_main_part(gp0, gx0, pv, x_ref, gr_ref, ber_ref, wr2_ref,
                       br2_ref, wu_ref, bu_ref, u_scr, s2_scr, st_ref, i)

    @pl.when(jnp.logical_and(ph == 1, i >= nh))
    def _():
        _agg_main_part(gp1, gx1, pv, x_ref, gr_ref, ber_ref, wr2_ref,
                       br2_ref, wu_ref, bu_ref, u_scr, s2_scr, st_ref, i)

    @pl.when(ph == 2)
    def _():
        cnt2 = jnp.float32(BN_ROWS)
        mu2 = s2_scr[0:1, :] / cnt2
        var2 = s2_scr[1:2, :] / cnt2 - mu2 * mu2
        scale2 = gu_ref[...] * lax.rsqrt(var2 + EPS)
        shift2 = beu_ref[...] - mu2 * scale2
        o_ref[...] = _gelu(u_scr[pl.ds(i * PB, PB), :] * scale2 + shift2)


def _agg(gpx0, gpx1, p, x2, g_r2, be_r2, wr2, br2, wu, bu, g_u2, be_u2):
    PB = 512
    nblk = BN_ROWS // PB
    nh = nblk // NH
    grid = (3, nblk)
    gp0m = lambda k, ph, i: (jnp.where(i < nh, k * nh + i, 0), 0)
    gp1m = lambda k, ph, i: (jnp.where(i >= nh, k * nh + i - nh, 0), 0)
    gx0m = lambda k, ph, i: (
        jnp.where(jnp.logical_and(ph == 1, i < nh), k * nh + i, 0), 1)
    gx1m = lambda k, ph, i: (
        jnp.where(jnp.logical_and(ph == 1, i >= nh), k * nh + i - nh, 0), 1)
    mk = lambda f, k: pl.BlockSpec((PB, D2), functools.partial(f, k))
    gp_specs = ([mk(gp0m, k) for k in range(K)]
                + [mk(gp1m, k) for k in range(K)])
    gx_specs = ([mk(gx0m, k) for k in range(K)]
                + [mk(gx1m, k) for k in range(K)])
    return pl.pallas_call(
        _agg_body,
        grid=grid,
        in_specs=gp_specs + gx_specs + [
            pl.BlockSpec((PB, D2), lambda ph, i: (i, 0)),
            pl.BlockSpec((PB, D), lambda ph, i: (jnp.where(ph == 0, 0, i), 0)),
            pl.BlockSpec((1, D2), lambda ph, i: (0, 0)),
            pl.BlockSpec((1, D2), lambda ph, i: (0, 0)),
            pl.BlockSpec((1, D2), lambda ph, i: (0, 0)),
            pl.BlockSpec((1, 1), lambda ph, i: (0, 0)),
            pl.BlockSpec((OUT, 2 * D), lambda ph, i: (0, 0)),
            pl.BlockSpec((1, OUT), lambda ph, i: (0, 0)),
            pl.BlockSpec((1, OUT), lambda ph, i: (0, 0)),
            pl.BlockSpec((1, OUT), lambda ph, i: (0, 0)),
        ],
        out_specs=pl.BlockSpec((PB, OUT),
                               lambda ph, i: (jnp.where(ph == 2, i, 0), 0)),
        out_shape=jax.ShapeDtypeStruct((BN_ROWS, OUT), jnp.float32),
        scratch_shapes=[pltpu.VMEM((8, D2), jnp.float32),
                        pltpu.VMEM((BN_ROWS, OUT), jnp.float32),
                        pltpu.VMEM((8, OUT), jnp.float32)],
    )(*([gpx0] * K), *([gpx1] * K), *([gpx0] * K), *([gpx1] * K),
      p, x2, g_r2, be_r2, wr2, br2, wu, bu, g_u2, be_u2)


def kernel(x, W_ft, b_ft, W_r1, b_r1, g_r, be_r, W_r2, b_r2, W_u, b_u, g_u, be_u):
    x2 = x.reshape(BN_ROWS, D)

    xt, p, xx3, x_pack = _feat(x2, W_ft, b_ft.reshape(1, D), W_r1)

    xt3 = xt.reshape(B, N, D)
    idx0 = _knn(xt3, xx3, 0)
    gpx0 = _gather_sc(x_pack, idx0.reshape(-1))
    idx1 = _knn(xt3, xx3, 1)
    gpx1 = _gather_sc(x_pack, idx1.reshape(-1))

    out = _agg(gpx0, gpx1, p, x2,
               g_r.reshape(1, D2), be_r.reshape(1, D2),
               W_r2.reshape(1, D2), b_r2.reshape(1, 1),
               W_u, b_u.reshape(1, OUT),
               g_u.reshape(1, OUT), be_u.reshape(1, OUT))
    return out.reshape(B, N, OUT)


# speedup vs baseline: 12.9430x; 1.1401x over previous
import functools

import jax
import jax.numpy as jnp
from jax import lax
from jax.experimental import pallas as pl
from jax.experimental.pallas import tpu as pltpu
from jax.experimental.pallas import tpu_sc as plsc

B, N, D, K, OUT = 4, 2048, 256, 9, 256
D2 = D // 2
EPS = 1e-5
BN_ROWS = B * N
M_EDGES = B * N * K

SC_NC, SC_NS = 2, 16
SC_NW = SC_NC * SC_NS
EDGES_PER_W = M_EDGES // SC_NW
GCHUNK = 128
NCHUNKS = EDGES_PER_W // GCHUNK


def _gelu(v):
    return 0.5 * v * (1.0 + lax.erf(v * 0.7071067811865476))


def _round_bf16_bits(v):
    u = lax.bitcast_convert_type(v, jnp.int32)
    r = u + 32767 + jnp.bitwise_and(jnp.right_shift(u, 16), 1)
    return jnp.bitwise_and(r, jnp.int32(-65536))


def _feat_body(x_ref, wft_ref, bft_ref, wr1_ref, xt_ref, p_ref, xx_ref,
               xp_ref):
    x = x_ref[...]
    xt = lax.dot_general(
        x, wft_ref[...], (((1,), (1,)), ((), ())),
        preferred_element_type=jnp.float32) + bft_ref[...]
    xt_ref[...] = xt
    p_ref[...] = lax.dot_general(
        x, wr1_ref[...], (((1,), (1,)), ((), ())),
        preferred_element_type=jnp.float32)
    xx_ref[...] = jnp.sum(xt * xt, axis=1)[None, None, :]
    lo = lax.slice(x, (0, 0), (x.shape[0], D2))
    hi = lax.slice(x, (0, D2), (x.shape[0], D))
    xp_ref[...] = jnp.bitwise_or(
        lax.shift_right_logical(_round_bf16_bits(lo), 16),
        _round_bf16_bits(hi))


def _feat(x2, W_ft, b_ft, W_r1):
    RB = 512
    grid = (BN_ROWS // RB,)
    nb = N // RB
    return pl.pallas_call(
        _feat_body,
        grid=grid,
        in_specs=[
            pl.BlockSpec((RB, D), lambda i: (i, 0)),
            pl.BlockSpec((D, D), lambda i: (0, 0)),
            pl.BlockSpec((1, D), lambda i: (0, 0)),
            pl.BlockSpec((D2, D), lambda i: (0, 0)),
        ],
        out_specs=[
            pl.BlockSpec((RB, D), lambda i: (i, 0)),
            pl.BlockSpec((RB, D2), lambda i: (i, 0)),
            pl.BlockSpec((1, 1, RB), lambda i: (i // nb, 0, i % nb)),
            pl.BlockSpec((RB, D2), lambda i: (i, 0)),
        ],
        out_shape=[
            jax.ShapeDtypeStruct((BN_ROWS, D), jnp.float32),
            jax.ShapeDtypeStruct((BN_ROWS, D2), jnp.float32),
            jax.ShapeDtypeStruct((B, 1, N), jnp.float32),
            jax.ShapeDtypeStruct((BN_ROWS, D2), jnp.int32),
        ],
    )(x2, W_ft, b_ft, W_r1)


def _knn_body(a_ref, all_ref, xx_ref, idx_ref):
    RB = a_ref.shape[1]
    a = a_ref[0]
    ax = all_ref[0]
    xx_a = jnp.sum(a * a, axis=1, keepdims=True)
    xx_all = xx_ref[0]
    xy = lax.dot_general(a, ax, (((1,), (1,)), ((), ())),
                         preferred_element_type=jnp.float32)
    d = xx_a + xx_all - 2.0 * xy
    iota = lax.broadcasted_iota(jnp.int32, (RB, N), 1)
    self_col = (lax.broadcasted_iota(jnp.int32, (RB, 1), 0)
                + pl.program_id(1) * RB)
    d = jnp.where(iota == self_col, jnp.float32(jnp.inf), d)

    NG, Q = 16, 4
    L = N // NG
    INF = jnp.float32(jnp.inf)
    iota_l = lax.broadcasted_iota(jnp.int32, (RB, L), 1).astype(jnp.float32)
    vs = [jnp.full((RB, L), INF, jnp.float32) for _ in range(Q)]
    cs = [jnp.zeros((RB, L), jnp.float32) for _ in range(Q)]
    for g in range(NG):
        val = lax.slice(d, (0, g * L), (RB, (g + 1) * L))
        col = iota_l + jnp.float32(g * L)
        for q in range(Q):
            lt = val < vs[q]
            nv = jnp.where(lt, val, vs[q])
            nc = jnp.where(lt, col, cs[q])
            val = jnp.where(lt, vs[q], val)
            col = jnp.where(lt, cs[q], col)
            vs[q], cs[q] = nv, nc
    cols = []
    for _ in range(K):
        m = jnp.min(vs[0], axis=1, keepdims=True)
        am = jnp.min(jnp.where(vs[0] == m, cs[0], jnp.float32(N)), axis=1,
                     keepdims=True)
        cols.append(am)
        hit = cs[0] == am
        for q in range(Q - 1):
            vs[q] = jnp.where(hit, vs[q + 1], vs[q])
            cs[q] = jnp.where(hit, cs[q + 1], cs[q])
        vs[Q - 1] = jnp.where(hit, INF, vs[Q - 1])
        cs[Q - 1] = jnp.where(hit, jnp.float32(0), cs[Q - 1])
    idx = (jnp.concatenate(cols, axis=1).astype(jnp.int32)
           + pl.program_id(0) * N)
    idx_ref[...] = idx.T


def _knn(xt3, xx3):
    RB = 512
    grid = (B, N // RB)
    return pl.pallas_call(
        _knn_body,
        grid=grid,
        in_specs=[
            pl.BlockSpec((1, RB, D), lambda b, r: (b, r, 0)),
            pl.BlockSpec((1, N, D), lambda b, r: (b, 0, 0)),
            pl.BlockSpec((1, 1, N), lambda b, r: (b, 0, 0)),
        ],
        out_specs=pl.BlockSpec((K, RB), lambda b, r: (0, b * (N // RB) + r)),
        out_shape=jax.ShapeDtypeStruct((K, BN_ROWS), jnp.int32),
    )(xt3, xt3, xx3)


def _gather_sc_body(p_hbm, x_hbm, idx_hbm, gp_hbm, gx_hbm,
                    idx_a, idx_b, prow_a, prow_b, xrow_a, xrow_b,
                    sem_ga, sem_gb, sem_wa, sem_wb):
    wid = lax.axis_index("s") * SC_NC + lax.axis_index("c")
    bufs = [(idx_a, prow_a, xrow_a, sem_ga, sem_wa),
            (idx_b, prow_b, xrow_b, sem_gb, sem_wb)]

    def start_gather(c):
        idx_v, prow_v, xrow_v, sem_g, _ = bufs[c % 2]
        base = wid * EDGES_PER_W + c * GCHUNK
        pltpu.sync_copy(idx_hbm.at[pl.ds(base, GCHUNK)], idx_v)
        return (pltpu.async_copy(p_hbm.at[idx_v], prow_v, sem_g),
                pltpu.async_copy(x_hbm.at[idx_v], xrow_v, sem_g))

    def start_write(c):
        _, prow_v, xrow_v, _, sem_w = bufs[c % 2]
        base = wid * EDGES_PER_W + c * GCHUNK
        return (pltpu.async_copy(prow_v, gp_hbm.at[pl.ds(base, GCHUNK)], sem_w),
                pltpu.async_copy(xrow_v, gx_hbm.at[pl.ds(base, GCHUNK)], sem_w))

    pending_g = {0: start_gather(0)}
    pending_w = {}
    for c in range(NCHUNKS):
        if c + 1 < NCHUNKS:
            if c - 1 >= 0:
                for h in pending_w.pop(c - 1):
                    h.wait()
            pending_g[c + 1] = start_gather(c + 1)
        for h in pending_g.pop(c):
            h.wait()
        pending_w[c] = start_write(c)
    for c in (NCHUNKS - 2, NCHUNKS - 1):
        for h in pending_w.pop(c, ()):
            h.wait()


@functools.lru_cache(maxsize=1)
def _make_gather_sc():
    return pl.kernel(
        _gather_sc_body,
        out_type=[
            jax.ShapeDtypeStruct((M_EDGES, D2), jnp.float32),
            jax.ShapeDtypeStruct((M_EDGES, D2), jnp.int32),
        ],
        mesh=plsc.VectorSubcoreMesh(core_axis_name="c", subcore_axis_name="s",
                                    num_cores=SC_NC, num_subcores=SC_NS),
        scratch_types=[
            pltpu.VMEM((GCHUNK,), jnp.int32),
            pltpu.VMEM((GCHUNK,), jnp.int32),
            pltpu.VMEM((GCHUNK, D2), jnp.float32),
            pltpu.VMEM((GCHUNK, D2), jnp.float32),
            pltpu.VMEM((GCHUNK, D2), jnp.int32),
            pltpu.VMEM((GCHUNK, D2), jnp.int32),
            pltpu.SemaphoreType.DMA,
            pltpu.SemaphoreType.DMA,
            pltpu.SemaphoreType.DMA,
            pltpu.SemaphoreType.DMA,
        ],
    )


def _gather_sc(p, x_bf3, idx_flat):
    return _make_gather_sc()(p, x_bf3, idx_flat)


def _stats_body(*refs):
    gp_refs = refs[:K]
    p_ref = refs[K]
    out_ref = refs[K + 1]
    pv = p_ref[...]
    s1 = jnp.zeros((1, D2), jnp.float32)
    s2 = jnp.zeros((1, D2), jnp.float32)
    for k in range(K):
        y = gp_refs[k][...] - pv
        s1 = s1 + jnp.sum(y, axis=0, keepdims=True)
        s2 = s2 + jnp.sum(y * y, axis=0, keepdims=True)

    @pl.when(pl.program_id(0) == 0)
    def _():
        out_ref[...] = jnp.zeros_like(out_ref)

    out_ref[0:1, :] += s1
    out_ref[1:2, :] += s2


def _stats(gp_t, p):
    PB = 512
    nblk = BN_ROWS // PB
    grid = (nblk,)
    gp_specs = [
        pl.BlockSpec((PB, D2), functools.partial(lambda k, i: (k * nblk + i, 0), k))
        for k in range(K)
    ]
    return pl.pallas_call(
        _stats_body,
        grid=grid,
        in_specs=gp_specs + [pl.BlockSpec((PB, D2), lambda i: (i, 0))],
        out_specs=pl.BlockSpec((8, D2), lambda i: (0, 0)),
        out_shape=jax.ShapeDtypeStruct((8, D2), jnp.float32),
    )(*([gp_t] * K), p)


def _agg_body(*refs):
    gp_refs = refs[:K]
    gx_refs = refs[K:2 * K]
    (p_ref, x_ref, sums_ref, gr_ref, ber_ref, wr2_ref, br2_ref,
     wu_ref, bu_ref, u_ref, s2_ref) = refs[2 * K:]
    cnt = jnp.float32(M_EDGES)
    mu = sums_ref[0:1, :] / cnt
    var = sums_ref[1:2, :] / cnt - mu * mu
    scale = gr_ref[...] * lax.rsqrt(var + EPS)
    shift = ber_ref[...] - mu * scale
    wr2 = wr2_ref[...]
    br2 = br2_ref[...]
    wu = wu_ref[...]
    wux = lax.slice(wu, (0, 0), (OUT, D))
    wua = lax.slice(wu, (0, D), (OUT, 2 * D))

    pv = p_ref[...]
    xv = x_ref[...]
    PB = xv.shape[0]
    agg_e = jnp.zeros((PB, D2), jnp.float32)
    agg_o = jnp.zeros((PB, D2), jnp.float32)
    wsum = jnp.zeros((PB, 1), jnp.float32)
    for k in range(K):
        y = gp_refs[k][...] - pv
        h = _gelu(y * scale + shift)
        z = jnp.sum(h * wr2, axis=1, keepdims=True) + br2
        w = 1.0 / (1.0 + jnp.exp(-z))
        gxi = gx_refs[k][...]
        fe = lax.bitcast_convert_type(jnp.left_shift(gxi, 16), jnp.float32)
        fo = lax.bitcast_convert_type(
            jnp.bitwise_and(gxi, jnp.int32(-65536)), jnp.float32)
        agg_e = agg_e + w * fe
        agg_o = agg_o + w * fo
        wsum = wsum + w
    agg = jnp.concatenate([agg_e, agg_o], axis=1) - wsum * xv

    u = (lax.dot_general(xv, wux, (((1,), (1,)), ((), ())),
                         preferred_element_type=jnp.float32)
         + lax.dot_general(agg, wua, (((1,), (1,)), ((), ())),
                           preferred_element_type=jnp.float32)
         + bu_ref[...])
    u_ref[...] = u

    @pl.when(pl.program_id(0) == 0)
    def _():
        s2_ref[...] = jnp.zeros_like(s2_ref)

    s2_ref[0:1, :] += jnp.sum(u, axis=0, keepdims=True)
    s2_ref[1:2, :] += jnp.sum(u * u, axis=0, keepdims=True)


def _agg(gp_t, gx_t, p, x2, sums, g_r2, be_r2, wr2, br2, wu, bu):
    PB = 512
    nblk = BN_ROWS // PB
    grid = (nblk,)
    kmap = lambda k, i: (k * nblk + i, 0)
    gp_specs = [pl.BlockSpec((PB, D2), functools.partial(kmap, k))
                for k in range(K)]
    gx_specs = [pl.BlockSpec((PB, D2), functools.partial(kmap, k))
                for k in range(K)]
    return pl.pallas_call(
        _agg_body,
        grid=grid,
        in_specs=gp_specs + gx_specs + [
            pl.BlockSpec((PB, D2), lambda i: (i, 0)),
            pl.BlockSpec((PB, D), lambda i: (i, 0)),
            pl.BlockSpec((8, D2), lambda i: (0, 0)),
            pl.BlockSpec((1, D2), lambda i: (0, 0)),
            pl.BlockSpec((1, D2), lambda i: (0, 0)),
            pl.BlockSpec((1, D2), lambda i: (0, 0)),
            pl.BlockSpec((1, 1), lambda i: (0, 0)),
            pl.BlockSpec((OUT, 2 * D), lambda i: (0, 0)),
            pl.BlockSpec((1, OUT), lambda i: (0, 0)),
        ],
        out_specs=[
            pl.BlockSpec((PB, OUT), lambda i: (i, 0)),
            pl.BlockSpec((8, OUT), lambda i: (0, 0)),
        ],
        out_shape=[
            jax.ShapeDtypeStruct((BN_ROWS, OUT), jnp.float32),
            jax.ShapeDtypeStruct((8, OUT), jnp.float32),
        ],
    )(*([gp_t] * K), *([gx_t] * K), p, x2, sums, g_r2, be_r2, wr2, br2,
      wu, bu)


def _final_body(u_ref, s2_ref, gu_ref, beu_ref, o_ref):
    cnt = jnp.float32(BN_ROWS)
    mu = s2_ref[0:1, :] / cnt
    var = s2_ref[1:2, :] / cnt - mu * mu
    scale = gu_ref[...] * lax.rsqrt(var + EPS)
    shift = beu_ref[...] - mu * scale
    o_ref[...] = _gelu(u_ref[...] * scale + shift)


def _final(u, s2, g_u2, be_u2):
    RB = 512
    grid = (BN_ROWS // RB,)
    return pl.pallas_call(
        _final_body,
        grid=grid,
        in_specs=[
            pl.BlockSpec((RB, OUT), lambda i: (i, 0)),
            pl.BlockSpec((8, OUT), lambda i: (0, 0)),
            pl.BlockSpec((1, OUT), lambda i: (0, 0)),
            pl.BlockSpec((1, OUT), lambda i: (0, 0)),
        ],
        out_specs=pl.BlockSpec((RB, OUT), lambda i: (i, 0)),
        out_shape=jax.ShapeDtypeStruct((BN_ROWS, OUT), jnp.float32),
    )(u, s2, g_u2, be_u2)


def kernel(x, W_ft, b_ft, W_r1, b_r1, g_r, be_r, W_r2, b_r2, W_u, b_u, g_u, be_u):
    x2 = x.reshape(BN_ROWS, D)

    xt, p, xx3, x_pack = _feat(x2, W_ft, b_ft.reshape(1, D), W_r1)

    idx9 = _knn(xt.reshape(B, N, D), xx3)
    idx_flat = idx9.reshape(-1)

    gp, gx = _gather_sc(p, x_pack, idx_flat)

    sums = _stats(gp, p)

    u, s2 = _agg(gp, gx, p, x2, sums,
                 g_r.reshape(1, D2), be_r.reshape(1, D2),
                 W_r2.reshape(1, D2), b_r2.reshape(1, 1),
                 W_u, b_u.reshape(1, OUT))

    out = _final(u, s2, g_u.reshape(1, OUT), be_u.reshape(1, OUT))
    return out.reshape(B, N, OUT)
